# baseline (speedup 1.0000x reference)
import functools

import jax
import jax.numpy as jnp
from jax import lax
from jax.experimental import pallas as pl
from jax.experimental.pallas import tpu as pltpu
from jax.experimental.pallas import tpu_sc as plsc

B = 128
C = 5
HW = 256
PLANE = C * HW
NSUB = 16
SPB = B // NSUB
L = 16
ROWW = 128
NG = 32
INV_S = 1.0 / 32.0
NOOBJ = 0.5
COORD = 5.0


def _sqrt16(a):
    i = lax.bitcast_convert_type(a, jnp.int32)
    i = jnp.int32(0x5F3759DF) - lax.shift_right_logical(i, 1)
    y = lax.bitcast_convert_type(i, jnp.float32)
    for _ in range(4):
        y = y * (1.5 - 0.5 * a * y * y)
    return a * y


def _sc_loss(preds_flat, preds_tbl, x, y, w, h, label):
    mesh = plsc.VectorSubcoreMesh(
        core_axis_name="c", subcore_axis_name="s", num_cores=1)

    @functools.partial(
        pl.kernel,
        mesh=mesh,
        compiler_params=pltpu.CompilerParams(needs_layout_passes=False),
        out_type=jax.ShapeDtypeStruct((L,), jnp.float32),
        scratch_types=[
            pltpu.VMEM((SPB * HW,), jnp.float32),
            pltpu.VMEM((NG, ROWW), jnp.float32),
            pltpu.VMEM((NG,), jnp.int32),
            pltpu.VMEM((L,), jnp.int32),
            pltpu.VMEM((SPB,), jnp.float32),
            pltpu.VMEM((SPB,), jnp.float32),
            pltpu.VMEM((SPB,), jnp.float32),
            pltpu.VMEM((SPB,), jnp.float32),
            pltpu.VMEM((SPB,), jnp.float32),
            pltpu.VMEM((L,), jnp.float32),
            pltpu.VMEM_SHARED((L,), jnp.float32),
            pltpu.VMEM((L,), jnp.float32),
            pltpu.SemaphoreType.DMA,
            pltpu.SemaphoreType.DMA,
            pltpu.SemaphoreType.DMA,
        ],
    )
    def k(preds_hbm, tbl_hbm, x_hbm, y_hbm, w_hbm, h_hbm, label_hbm, out_hbm,
          bufc, gbuf, idxv, cellv, xv, yv, wv, hv, lv, pvec, sacc, outv,
          sem, sem2, semg):
        sid = lax.axis_index("s")
        base = sid * SPB

        @pl.when(sid == 0)
        def _():
            outv[...] = jnp.zeros((L,), jnp.float32)
            pltpu.sync_copy(outv, sacc)
        plsc.subcore_barrier()

        cps_conf = [
            pltpu.async_copy(preds_hbm.at[pl.ds((base + s) * PLANE, HW)],
                             bufc.at[pl.ds(s * HW, HW)], sem)
            for s in range(SPB)
        ]
        cps = [
            pltpu.async_copy(x_hbm.at[pl.ds(base, SPB)], xv, sem2),
            pltpu.async_copy(y_hbm.at[pl.ds(base, SPB)], yv, sem2),
            pltpu.async_copy(w_hbm.at[pl.ds(base, SPB)], wv, sem2),
            pltpu.async_copy(h_hbm.at[pl.ds(base, SPB)], hv, sem2),
            pltpu.async_copy(label_hbm.at[pl.ds(base, SPB)], lv, sem2),
        ]
        for cp in cps:
            cp.wait()

        lane = lax.iota(jnp.int32, L)
        sv = jnp.minimum(lane, SPB - 1)
        mask = lane < SPB

        xg = plsc.load_gather(xv, [sv])
        yg = plsc.load_gather(yv, [sv])
        wg = plsc.load_gather(wv, [sv])
        hg = plsc.load_gather(hv, [sv])
        lg = plsc.load_gather(lv, [sv])

        tx = (lax.rem(xg, 32.0) * INV_S - 0.5) * lg
        ty = (lax.rem(yg, 32.0) * INV_S - 0.5) * lg
        xi = (xg * INV_S).astype(jnp.int32)
        yi = (yg * INV_S).astype(jnp.int32)
        cell = xi * 16 + yi
        stw = _sqrt16(wg * INV_S)
        sth = _sqrt16(hg * INV_S)

        cellv[...] = cell
        sm = lane & 7
        cg = plsc.load_gather(cellv, [sm])
        kv1 = 1 + lax.shift_right_logical(lane, 3)
        kv2 = kv1 + 2
        f1 = (base + sm) * PLANE + kv1 * HW + cg
        f2 = (base + sm) * PLANE + kv2 * HW + cg
        idxv[pl.ds(0, L)] = lax.shift_right_logical(f1, 7)
        idxv[pl.ds(L, L)] = lax.shift_right_logical(f2, 7)
        cp_g = pltpu.async_copy(tbl_hbm.at[idxv], gbuf, semg)

        for cp in cps_conf:
            cp.wait()
        a0 = jnp.zeros((L,), jnp.float32)
        a1 = jnp.zeros((L,), jnp.float32)
        a2 = jnp.zeros((L,), jnp.float32)
        a3 = jnp.zeros((L,), jnp.float32)
        for s in range(SPB):
            for i in range(0, HW // L, 4):
                v0 = bufc[pl.ds(s * HW + i * L, L)]
                v1 = bufc[pl.ds(s * HW + (i + 1) * L, L)]
                v2 = bufc[pl.ds(s * HW + (i + 2) * L, L)]
                v3 = bufc[pl.ds(s * HW + (i + 3) * L, L)]
                a0 = a0 + v0 * v0
                a1 = a1 + v1 * v1
                a2 = a2 + v2 * v2
                a3 = a3 + v3 * v3
        acc = (a0 + a1) + (a2 + a3)

        c0 = plsc.load_gather(bufc, [sv * HW + cell])

        cp_g.wait()
        col = cell & 127
        c1 = plsc.load_gather(gbuf, [sv, col])
        c2 = plsc.load_gather(gbuf, [8 + sv, col])
        c3 = plsc.load_gather(gbuf, [16 + sv, col])
        c4 = plsc.load_gather(gbuf, [24 + sv, col])

        dw = stw - _sqrt16(c3)
        dh = sth - _sqrt16(c4)
        term = ((1.0 - c0) * (1.0 - c0) - NOOBJ * c0 * c0
                + COORD * ((tx - c1) * (tx - c1)
                           + (ty - c2) * (ty - c2)
                           + dw * dw + dh * dh))
        partial = NOOBJ * acc + jnp.where(mask, term, 0.0)

        pvec[...] = jnp.full((L,), jnp.sum(partial), jnp.float32)
        pltpu.sync_copy(pvec, sacc.at[jnp.arange(L, dtype=jnp.int32)], add=True)
        plsc.subcore_barrier()

        @pl.when(sid == 0)
        def _():
            pltpu.sync_copy(sacc, out_hbm)

    return k(preds_flat, preds_tbl, x, y, w, h, label)


def kernel(preds, x, y, w, h, label):
    flat = preds.reshape(-1)
    out = _sc_loss(flat, flat.reshape(-1, ROWW), x, y, w, h, label)
    return out[0]


# speedup vs baseline: 1.1165x; 1.1165x over previous
import functools

import jax
import jax.numpy as jnp
from jax import lax
from jax.experimental import pallas as pl
from jax.experimental.pallas import tpu as pltpu
from jax.experimental.pallas import tpu_sc as plsc

B = 128
C = 5
HW = 256
PLANE = C * HW
NSUB = 16
SPB = B // NSUB
SLAB = SPB * PLANE
L = 16
INV_S = 1.0 / 32.0
NOOBJ = 0.5
COORD = 5.0


def _sqrt16(a):
    i = lax.bitcast_convert_type(a, jnp.int32)
    i = jnp.int32(0x5F3759DF) - lax.shift_right_logical(i, 1)
    y = lax.bitcast_convert_type(i, jnp.float32)
    for _ in range(4):
        y = y * (1.5 - 0.5 * a * y * y)
    return a * y


def _sc_loss(preds_flat, x, y, w, h, label):
    mesh = plsc.VectorSubcoreMesh(
        core_axis_name="c", subcore_axis_name="s", num_cores=1)

    @functools.partial(
        pl.kernel,
        mesh=mesh,
        compiler_params=pltpu.CompilerParams(needs_layout_passes=False),
        out_type=jax.ShapeDtypeStruct((L,), jnp.float32),
        scratch_types=[
            pltpu.VMEM((SLAB,), jnp.float32),
            pltpu.VMEM((SPB,), jnp.float32),
            pltpu.VMEM((SPB,), jnp.float32),
            pltpu.VMEM((SPB,), jnp.float32),
            pltpu.VMEM((SPB,), jnp.float32),
            pltpu.VMEM((SPB,), jnp.float32),
            pltpu.VMEM((L,), jnp.float32),
            pltpu.VMEM_SHARED((L,), jnp.float32),
            pltpu.VMEM((L,), jnp.float32),
            pltpu.SemaphoreType.DMA,
            pltpu.SemaphoreType.DMA,
        ],
    )
    def k(preds_hbm, x_hbm, y_hbm, w_hbm, h_hbm, label_hbm, out_hbm,
          buf, xv, yv, wv, hv, lv, pvec, sacc, outv, sem, sem2):
        sid = lax.axis_index("s")
        base = sid * SPB

        @pl.when(sid == 0)
        def _():
            outv[...] = jnp.zeros((L,), jnp.float32)
            pltpu.sync_copy(outv, sacc)
        plsc.subcore_barrier()

        cp_slab = pltpu.async_copy(
            preds_hbm.at[pl.ds(sid * SLAB, SLAB)], buf, sem)
        cps = [
            pltpu.async_copy(x_hbm.at[pl.ds(base, SPB)], xv, sem2),
            pltpu.async_copy(y_hbm.at[pl.ds(base, SPB)], yv, sem2),
            pltpu.async_copy(w_hbm.at[pl.ds(base, SPB)], wv, sem2),
            pltpu.async_copy(h_hbm.at[pl.ds(base, SPB)], hv, sem2),
            pltpu.async_copy(label_hbm.at[pl.ds(base, SPB)], lv, sem2),
        ]
        for cp in cps:
            cp.wait()

        lane = lax.iota(jnp.int32, L)
        sv = jnp.minimum(lane, SPB - 1)
        mask = lane < SPB

        xg = plsc.load_gather(xv, [sv])
        yg = plsc.load_gather(yv, [sv])
        wg = plsc.load_gather(wv, [sv])
        hg = plsc.load_gather(hv, [sv])
        lg = plsc.load_gather(lv, [sv])

        tx = (lax.rem(xg, 32.0) * INV_S - 0.5) * lg
        ty = (lax.rem(yg, 32.0) * INV_S - 0.5) * lg
        xi = (xg * INV_S).astype(jnp.int32)
        yi = (yg * INV_S).astype(jnp.int32)
        cell = xi * 16 + yi
        stw = _sqrt16(wg * INV_S)
        sth = _sqrt16(hg * INV_S)

        a0 = jnp.zeros((L,), jnp.float32)
        a1 = jnp.zeros((L,), jnp.float32)
        a2 = jnp.zeros((L,), jnp.float32)
        a3 = jnp.zeros((L,), jnp.float32)
        cp_slab.wait()
        for s in range(SPB):
            for i in range(0, HW // L, 4):
                v0 = buf[pl.ds(s * PLANE + i * L, L)]
                v1 = buf[pl.ds(s * PLANE + (i + 1) * L, L)]
                v2 = buf[pl.ds(s * PLANE + (i + 2) * L, L)]
                v3 = buf[pl.ds(s * PLANE + (i + 3) * L, L)]
                a0 = a0 + v0 * v0
                a1 = a1 + v1 * v1
                a2 = a2 + v2 * v2
                a3 = a3 + v3 * v3
        acc = (a0 + a1) + (a2 + a3)

        cbase = sv * PLANE + cell
        c0 = plsc.load_gather(buf, [cbase])
        c1 = plsc.load_gather(buf, [cbase + HW])
        c2 = plsc.load_gather(buf, [cbase + 2 * HW])
        c3 = plsc.load_gather(buf, [cbase + 3 * HW])
        c4 = plsc.load_gather(buf, [cbase + 4 * HW])

        dw = stw - _sqrt16(c3)
        dh = sth - _sqrt16(c4)
        term = ((1.0 - c0) * (1.0 - c0) - NOOBJ * c0 * c0
                + COORD * ((tx - c1) * (tx - c1)
                           + (ty - c2) * (ty - c2)
                           + dw * dw + dh * dh))
        partial = NOOBJ * acc + jnp.where(mask, term, 0.0)

        pvec[...] = jnp.full((L,), jnp.sum(partial), jnp.float32)
        pltpu.sync_copy(pvec, sacc.at[jnp.arange(L, dtype=jnp.int32)], add=True)
        plsc.subcore_barrier()

        @pl.when(sid == 0)
        def _():
            pltpu.sync_copy(sacc, out_hbm)

    return k(preds_flat, x, y, w, h, label)


def kernel(preds, x, y, w, h, label):
    out = _sc_loss(preds.reshape(-1), x, y, w, h, label)
    return out[0]
